# baseline (speedup 1.0000x reference)
import functools

import jax
import jax.numpy as jnp
from jax import lax
from jax.experimental import pallas as pl
from jax.experimental.pallas import tpu as pltpu
from jax.experimental.pallas import tpu_sc as plsc

N = 10000
D = 128
DH = 256
E = 320000
NC = 2
NS = 16
K = 128
EPS_SUB = 20480
NCH = EPS_SUB // K
NACC = N + 8
FL_W = 640
FL_S = 624
ZR = 64
ZCH = FL_W // ZR
IDXCH = 5
W = 32
NBUF = 5
GLEAD = 3
SLAG = 2
RB = 2000
NG = N // RB


def _fill(ref, rows, cols, value):
    per_row = cols // 16

    def body(i, _):
        r = i // per_row
        c = (i % per_row) * 16
        ref[r, pl.ds(c, 16)] = jnp.full((16,), value, jnp.float32)
        return 0

    lax.fori_loop(0, rows * per_row, body, 0)


def _xform_idx(idxs_v, idxg_v, stride, off):
    per_row = K // 16

    def body(i, _):
        r = i // per_row
        c = (i % per_row) * 16
        idxg_v[r, pl.ds(c, 16)] = idxs_v[r, pl.ds(c, 16)] * stride + off
        return 0

    lax.fori_loop(0, NCH * per_row, body, 0)


def _zero_acc(acc, zb, sid):
    def body(k, _):
        pltpu.sync_copy(zb, acc.at[pl.ds(sid * FL_S + k * ZR, ZR)])
        return 0

    lax.fori_loop(0, ZCH, body, 0)


def _flush_acc(acc, out, cid, t, sid):
    def body(k, _):
        r0 = sid * FL_S + k * ZR
        pltpu.sync_copy(acc.at[pl.ds(r0, ZR)],
                        out.at[cid, t, pl.ds(r0, ZR)])
        return 0

    lax.fori_loop(0, ZCH, body, 0)


def _sc_seg_body(n_tasks, with_counts, stride, core_span, task_span, *refs):
    (src_all, dst_all, table, out, idxs_v, idxd_v, idxg_v, rows) = refs[:8]
    rest = refs[8:]
    if with_counts:
        ones_v, zb, acc = rest[:3]
        rest = rest[3:]
    else:
        zb, acc = rest[:2]
        rest = rest[2:]
        ones_v = None
    sem_g, sem_s = rest[:2]

    cid = lax.axis_index("c")
    sid = lax.axis_index("s")

    _fill(zb, ZR, W, 0.0)
    if with_counts:
        _fill(ones_v, K, W, 1.0)

    def load_idx(k, _):
        r = k * (NCH // IDXCH)
        sl = pl.ds(r, NCH // IDXCH)
        pltpu.sync_copy(src_all.at[cid, sid, sl], idxs_v.at[sl])
        pltpu.sync_copy(dst_all.at[cid, sid, sl], idxd_v.at[sl])
        return 0

    lax.fori_loop(0, IDXCH, load_idx, 0)

    def run_task(t, gather):
        _zero_acc(acc, zb, sid)
        plsc.subcore_barrier()

        if gather:
            _xform_idx(idxs_v, idxg_v, stride,
                       cid * core_span + t * task_span)

            def prologue(b, _):
                pltpu.async_copy(table.at[idxg_v.at[b]], rows.at[b],
                                 sem_g.at[b])
                return 0

            lax.fori_loop(0, GLEAD, prologue, 0)

            def body(j, _):
                b = j % NBUF
                pltpu.make_async_copy(table.at[idxg_v.at[j]],
                                      rows.at[b], sem_g.at[b]).wait()
                pltpu.async_copy(rows.at[b], acc.at[idxd_v.at[j]],
                                 sem_s.at[b], add=True)

                @pl.when(j >= SLAG)
                def _():
                    b2 = (j - SLAG) % NBUF
                    pltpu.make_async_copy(
                        rows.at[b2], acc.at[idxd_v.at[j - SLAG]],
                        sem_s.at[b2]).wait()

                @pl.when(j + GLEAD < NCH)
                def _():
                    b3 = (j + GLEAD) % NBUF
                    pltpu.async_copy(table.at[idxg_v.at[j + GLEAD]],
                                     rows.at[b3], sem_g.at[b3])
                return 0

            lax.fori_loop(0, NCH, body, 0)

            def epilogue(jj, _):
                pltpu.make_async_copy(rows.at[jj % NBUF],
                                      acc.at[idxd_v.at[jj]],
                                      sem_s.at[jj % NBUF]).wait()
                return 0

            lax.fori_loop(NCH - SLAG, NCH, epilogue, 0)
        else:
            def body(j, _):
                pltpu.async_copy(ones_v, acc.at[idxd_v.at[j]],
                                 sem_s.at[0], add=True)

                @pl.when(j >= NBUF - 1)
                def _():
                    pltpu.make_async_copy(
                        ones_v, acc.at[idxd_v.at[j]], sem_s.at[0]).wait()
                return 0

            lax.fori_loop(0, NCH, body, 0)

            def drain(j, _):
                pltpu.make_async_copy(ones_v, acc.at[idxd_v.at[j]],
                                      sem_s.at[0]).wait()
                return 0

            lax.fori_loop(0, NBUF - 1, drain, 0)

        plsc.subcore_barrier()
        _flush_acc(acc, out, cid, t, sid)
        plsc.subcore_barrier()

    def task_body(t, _):
        run_task(t, True)
        return 0

    lax.fori_loop(0, n_tasks, task_body, 0)
    if with_counts:
        run_task(n_tasks, False)


def _make_sc_seg(n_tasks, with_counts, stride, core_span, task_span,
                 table_rows):
    f32 = jnp.float32
    n_out = n_tasks + (1 if with_counts else 0)
    scratch = [
        pltpu.VMEM((NCH, K), jnp.int32),
        pltpu.VMEM((NCH, K), jnp.int32),
        pltpu.VMEM((NCH, K), jnp.int32),
    ]
    scratch.append(pltpu.VMEM((NBUF, K, W), f32))
    if with_counts:
        scratch.append(pltpu.VMEM((K, W), f32))
    scratch.append(pltpu.VMEM((ZR, W), f32))
    scratch.append(pltpu.VMEM_SHARED((NACC, W), f32))
    scratch += [pltpu.SemaphoreType.DMA((NBUF,))] * 2
    return pl.kernel(
        functools.partial(_sc_seg_body, n_tasks, with_counts, stride,
                          core_span, task_span),
        out_type=jax.ShapeDtypeStruct((NC, n_out, N, W), f32),
        mesh=plsc.VectorSubcoreMesh(core_axis_name="c", subcore_axis_name="s"),
        compiler_params=pltpu.CompilerParams(use_tc_tiling_on_sc=False),
        scratch_types=scratch,
    )


def _tc_mm_body(n_s, s_ref, cnt_ref, x_ref, wn_ref, wr_ref, b_ref,
                z_ref, stats_ref, acc_ref):
    rcp = 1.0 / jnp.maximum(cnt_ref[0, 0, :, 0:1], 1.0)
    z = jnp.broadcast_to(b_ref[0], (RB, DH))
    for q in range(n_s):
        z = z + jnp.dot(s_ref[0, q] * rcp, wn_ref[0, q * W:(q + 1) * W, :],
                        preferred_element_type=jnp.float32)
    z = z + jnp.dot(x_ref[0], wr_ref[0], preferred_element_type=jnp.float32)
    z_ref[0] = z
    i = pl.program_id(1)

    @pl.when(i == 0)
    def _():
        acc_ref[...] = jnp.zeros_like(acc_ref)

    acc_ref[0:1, :] += jnp.sum(z, axis=0, keepdims=True)
    acc_ref[1:2, :] += jnp.sum(z * z, axis=0, keepdims=True)

    @pl.when(i == NG - 1)
    def _():
        stats_ref[0] = acc_ref[...]


def _tc_bn_body(z_ref, stats_ref, g_ref, be_ref, o_ref):
    m = stats_ref[0, 0:1, :] * (1.0 / N)
    v = stats_ref[0, 1:2, :] * (1.0 / N) - m * m
    z = z_ref[0]
    zn = g_ref[0] * (z - m) / jnp.sqrt(v + 1e-5) + be_ref[0]
    o_ref[0] = jnp.where(zn > 0, zn, jnp.exp(zn) - 1.0)


def _tc_dense(s_all, n_s, cnt_all, cnt_slot, x, x_flip, wn, wr, b, g, be,
              out_flip):
    f32 = jnp.float32
    xw = x.shape[-1]
    xsel = (lambda t: 1 - t) if x_flip else (lambda t: t)
    osel = (lambda t: 1 - t) if out_flip else (lambda t: t)
    z, stats = pl.pallas_call(
        functools.partial(_tc_mm_body, n_s),
        grid=(2, NG),
        in_specs=[
            pl.BlockSpec((1, n_s, RB, W), lambda t, i: (t, 0, i, 0)),
            pl.BlockSpec((1, 1, RB, W), lambda t, i: (t, cnt_slot, i, 0)),
            pl.BlockSpec((1, RB, xw), lambda t, i: (xsel(t), i, 0)),
            pl.BlockSpec((1, n_s * W, DH), lambda t, i: (t, 0, 0)),
            pl.BlockSpec((1, xw, DH), lambda t, i: (t, 0, 0)),
            pl.BlockSpec((1, 1, DH), lambda t, i: (t, 0, 0)),
        ],
        out_specs=(pl.BlockSpec((1, RB, DH), lambda t, i: (t, i, 0)),
                   pl.BlockSpec((1, 2, DH), lambda t, i: (t, 0, 0))),
        out_shape=(jax.ShapeDtypeStruct((2, N, DH), f32),
                   jax.ShapeDtypeStruct((2, 2, DH), f32)),
        scratch_shapes=[pltpu.VMEM((2, DH), f32)],
    )(s_all, cnt_all, x, wn, wr, b)
    return pl.pallas_call(
        _tc_bn_body,
        grid=(2, NG),
        in_specs=[
            pl.BlockSpec((1, RB, DH), lambda t, i: (t, i, 0)),
            pl.BlockSpec((1, 2, DH), lambda t, i: (t, 0, 0)),
            pl.BlockSpec((1, 1, DH), lambda t, i: (t, 0, 0)),
            pl.BlockSpec((1, 1, DH), lambda t, i: (t, 0, 0)),
        ],
        out_specs=pl.BlockSpec((1, RB, DH), lambda t, i: (osel(t), i, 0)),
        out_shape=jax.ShapeDtypeStruct((2, N, DH), f32),
    )(z, stats, g, be)


def kernel(x_user, x_item, edge_index_ui, edge_index_iu,
           W1r_ui, W1n_ui, b1_ui, W1r_iu, W1n_iu, b1_iu,
           W2r_ui, W2n_ui, b2_ui, W2r_iu, W2n_iu, b2_iu,
           g1_u, be1_u, g1_i, be1_i, g2_u, be2_u, g2_i, be2_i):
    ei_ui = edge_index_ui.astype(jnp.int32)
    ei_iu = edge_index_iu.astype(jnp.int32)

    pad_n = EPS_SUB - E // NS

    def shard(row, fill):
        p = jnp.full((NS, pad_n), fill, jnp.int32)
        return jnp.concatenate([row.reshape(NS, E // NS), p], axis=1)

    src_all = jnp.stack([shard(ei_iu[0], 0), shard(ei_ui[0], 0)])
    src_all = src_all.reshape(NC, NS, NCH, K)
    dst_all = jnp.stack([shard(ei_iu[1], N), shard(ei_ui[1], N)])
    dst_all = dst_all.reshape(NC, NS, NCH, K)

    stk = lambda a, bb: jnp.stack([a, bb])
    row2 = lambda a, bb: jnp.stack([a, bb]).reshape(2, 1, DH)

    NT1 = D // W
    x_flat1 = jnp.concatenate([x_item, x_user], axis=0).reshape(-1, W)
    sc1 = _make_sc_seg(NT1, True, NT1, NT1 * N, 1, 0)(src_all, dst_all,
                                                      x_flat1)

    h1 = _tc_dense(sc1, NT1, sc1, NT1, stk(x_user, x_item), False,
                   stk(W1n_iu, W1n_ui), stk(W1r_iu, W1r_ui),
                   row2(b1_iu, b1_ui), row2(g1_u, g1_i),
                   row2(be1_u, be1_i), True)

    NT2 = DH // W
    sc2 = _make_sc_seg(NT2, False, NT2, NT2 * N, 1, 0)(
        src_all, dst_all, h1.reshape(-1, W))

    h2 = _tc_dense(sc2, NT2, sc1, NT1, h1, True,
                   stk(W2n_iu, W2n_ui), stk(W2r_iu, W2r_ui),
                   row2(b2_iu, b2_ui), row2(g2_u, g2_i),
                   row2(be2_u, be2_i), False)

    return (x_user, x_item, h1[1], h1[0], h2[0], h2[1])


# speedup vs baseline: 1.3674x; 1.3674x over previous
import functools

import jax
import jax.numpy as jnp
from jax import lax
from jax.experimental import pallas as pl
from jax.experimental.pallas import tpu as pltpu
from jax.experimental.pallas import tpu_sc as plsc

N = 10000
D = 128
DH = 256
E = 320000
NC = 2
NS = 16
K = 64
EPS_SUB = 20160
NCH = EPS_SUB // K
NACC = N + 8
FL_W = 640
FL_S = 624
ZR = 16
ZCH = FL_W // ZR
IDXCH = 15
W = 64
NBUF = 5
GLEAD = 3
SLAG = 2
RB = 2000
NG = N // RB


def _fill(ref, rows, cols, value):
    per_row = cols // 16

    def body(i, _):
        r = i // per_row
        c = (i % per_row) * 16
        ref[r, pl.ds(c, 16)] = jnp.full((16,), value, jnp.float32)
        return 0

    lax.fori_loop(0, rows * per_row, body, 0)


def _xform_idx(idxs_v, idxg_v, stride, off):
    per_row = K // 16

    def body(i, _):
        r = i // per_row
        c = (i % per_row) * 16
        idxg_v[r, pl.ds(c, 16)] = idxs_v[r, pl.ds(c, 16)] * stride + off
        return 0

    lax.fori_loop(0, NCH * per_row, body, 0)


def _zero_acc(acc, zb, sid):
    def body(k, _):
        pltpu.sync_copy(zb, acc.at[pl.ds(sid * FL_S + k * ZR, ZR)])
        return 0

    lax.fori_loop(0, ZCH, body, 0)


def _flush_acc(acc, out, cid, t, sid):
    def body(k, _):
        r0 = sid * FL_S + k * ZR
        pltpu.sync_copy(acc.at[pl.ds(r0, ZR)],
                        out.at[cid, t, pl.ds(r0, ZR)])
        return 0

    lax.fori_loop(0, ZCH, body, 0)


def _sc_seg_body(n_tasks, with_counts, stride, core_span, task_span, *refs):
    (src_all, dst_all, table, out, idxs_v, idxd_v, idxg_v, rows) = refs[:8]
    rest = refs[8:]
    if with_counts:
        ones_v, zb, acc = rest[:3]
        rest = rest[3:]
    else:
        zb, acc = rest[:2]
        rest = rest[2:]
        ones_v = None
    sem_g, sem_s = rest[:2]

    cid = lax.axis_index("c")
    sid = lax.axis_index("s")

    _fill(zb, ZR, W, 0.0)
    if with_counts:
        _fill(ones_v, K, W, 1.0)

    def load_idx(k, _):
        r = k * (NCH // IDXCH)
        sl = pl.ds(r, NCH // IDXCH)
        pltpu.sync_copy(src_all.at[cid, sid, sl], idxs_v.at[sl])
        pltpu.sync_copy(dst_all.at[cid, sid, sl], idxd_v.at[sl])
        return 0

    lax.fori_loop(0, IDXCH, load_idx, 0)

    def run_task(t, gather):
        _zero_acc(acc, zb, sid)
        plsc.subcore_barrier()

        if gather:
            _xform_idx(idxs_v, idxg_v, stride,
                       cid * core_span + t * task_span)

            def prologue(b, _):
                pltpu.async_copy(table.at[idxg_v.at[b]], rows.at[b],
                                 sem_g.at[b])
                return 0

            lax.fori_loop(0, GLEAD, prologue, 0)

            def body(j, _):
                b = j % NBUF
                pltpu.make_async_copy(table.at[idxg_v.at[j]],
                                      rows.at[b], sem_g.at[b]).wait()
                pltpu.async_copy(rows.at[b], acc.at[idxd_v.at[j]],
                                 sem_s.at[b], add=True)

                @pl.when(j >= SLAG)
                def _():
                    b2 = (j - SLAG) % NBUF
                    pltpu.make_async_copy(
                        rows.at[b2], acc.at[idxd_v.at[j - SLAG]],
                        sem_s.at[b2]).wait()

                @pl.when(j + GLEAD < NCH)
                def _():
                    b3 = (j + GLEAD) % NBUF
                    pltpu.async_copy(table.at[idxg_v.at[j + GLEAD]],
                                     rows.at[b3], sem_g.at[b3])
                return 0

            lax.fori_loop(0, NCH, body, 0)

            def epilogue(jj, _):
                pltpu.make_async_copy(rows.at[jj % NBUF],
                                      acc.at[idxd_v.at[jj]],
                                      sem_s.at[jj % NBUF]).wait()
                return 0

            lax.fori_loop(NCH - SLAG, NCH, epilogue, 0)
        else:
            def body(j, _):
                pltpu.async_copy(ones_v, acc.at[idxd_v.at[j]],
                                 sem_s.at[0], add=True)

                @pl.when(j >= NBUF - 1)
                def _():
                    pltpu.make_async_copy(
                        ones_v, acc.at[idxd_v.at[j]], sem_s.at[0]).wait()
                return 0

            lax.fori_loop(0, NCH, body, 0)

            def drain(j, _):
                pltpu.make_async_copy(ones_v, acc.at[idxd_v.at[j]],
                                      sem_s.at[0]).wait()
                return 0

            lax.fori_loop(0, NBUF - 1, drain, 0)

        plsc.subcore_barrier()
        _flush_acc(acc, out, cid, t, sid)
        plsc.subcore_barrier()

    def task_body(t, _):
        run_task(t, True)
        return 0

    lax.fori_loop(0, n_tasks, task_body, 0)
    if with_counts:
        run_task(n_tasks, False)


def _make_sc_seg(n_tasks, with_counts, stride, core_span, task_span,
                 table_rows):
    f32 = jnp.float32
    n_out = n_tasks + (1 if with_counts else 0)
    scratch = [
        pltpu.VMEM((NCH, K), jnp.int32),
        pltpu.VMEM((NCH, K), jnp.int32),
        pltpu.VMEM((NCH, K), jnp.int32),
    ]
    scratch.append(pltpu.VMEM((NBUF, K, W), f32))
    if with_counts:
        scratch.append(pltpu.VMEM((K, W), f32))
    scratch.append(pltpu.VMEM((ZR, W), f32))
    scratch.append(pltpu.VMEM_SHARED((NACC, W), f32))
    scratch += [pltpu.SemaphoreType.DMA((NBUF,))] * 2
    return pl.kernel(
        functools.partial(_sc_seg_body, n_tasks, with_counts, stride,
                          core_span, task_span),
        out_type=jax.ShapeDtypeStruct((NC, n_out, N, W), f32),
        mesh=plsc.VectorSubcoreMesh(core_axis_name="c", subcore_axis_name="s"),
        compiler_params=pltpu.CompilerParams(use_tc_tiling_on_sc=False),
        scratch_types=scratch,
    )


def _tc_mm_body(n_s, s_ref, cnt_ref, x_ref, wn_ref, wr_ref, b_ref,
                z_ref, stats_ref, acc_ref):
    rcp = 1.0 / jnp.maximum(cnt_ref[0, 0, :, 0:1], 1.0)
    z = jnp.broadcast_to(b_ref[0], (RB, DH))
    for q in range(n_s):
        z = z + jnp.dot(s_ref[0, q] * rcp, wn_ref[0, q * W:(q + 1) * W, :],
                        preferred_element_type=jnp.float32)
    z = z + jnp.dot(x_ref[0], wr_ref[0], preferred_element_type=jnp.float32)
    z_ref[0] = z
    i = pl.program_id(1)

    @pl.when(i == 0)
    def _():
        acc_ref[...] = jnp.zeros_like(acc_ref)

    acc_ref[0:1, :] += jnp.sum(z, axis=0, keepdims=True)
    acc_ref[1:2, :] += jnp.sum(z * z, axis=0, keepdims=True)

    @pl.when(i == NG - 1)
    def _():
        stats_ref[0] = acc_ref[...]


def _tc_bn_body(z_ref, stats_ref, g_ref, be_ref, o_ref):
    m = stats_ref[0, 0:1, :] * (1.0 / N)
    v = stats_ref[0, 1:2, :] * (1.0 / N) - m * m
    z = z_ref[0]
    zn = g_ref[0] * (z - m) / jnp.sqrt(v + 1e-5) + be_ref[0]
    o_ref[0] = jnp.where(zn > 0, zn, jnp.exp(zn) - 1.0)


def _tc_dense(s_all, n_s, cnt_all, cnt_slot, x, x_flip, wn, wr, b, g, be,
              out_flip):
    f32 = jnp.float32
    xw = x.shape[-1]
    xsel = (lambda t: 1 - t) if x_flip else (lambda t: t)
    osel = (lambda t: 1 - t) if out_flip else (lambda t: t)
    z, stats = pl.pallas_call(
        functools.partial(_tc_mm_body, n_s),
        grid=(2, NG),
        in_specs=[
            pl.BlockSpec((1, n_s, RB, W), lambda t, i: (t, 0, i, 0)),
            pl.BlockSpec((1, 1, RB, W), lambda t, i: (t, cnt_slot, i, 0)),
            pl.BlockSpec((1, RB, xw), lambda t, i: (xsel(t), i, 0)),
            pl.BlockSpec((1, n_s * W, DH), lambda t, i: (t, 0, 0)),
            pl.BlockSpec((1, xw, DH), lambda t, i: (t, 0, 0)),
            pl.BlockSpec((1, 1, DH), lambda t, i: (t, 0, 0)),
        ],
        out_specs=(pl.BlockSpec((1, RB, DH), lambda t, i: (t, i, 0)),
                   pl.BlockSpec((1, 2, DH), lambda t, i: (t, 0, 0))),
        out_shape=(jax.ShapeDtypeStruct((2, N, DH), f32),
                   jax.ShapeDtypeStruct((2, 2, DH), f32)),
        scratch_shapes=[pltpu.VMEM((2, DH), f32)],
    )(s_all, cnt_all, x, wn, wr, b)
    return pl.pallas_call(
        _tc_bn_body,
        grid=(2, NG),
        in_specs=[
            pl.BlockSpec((1, RB, DH), lambda t, i: (t, i, 0)),
            pl.BlockSpec((1, 2, DH), lambda t, i: (t, 0, 0)),
            pl.BlockSpec((1, 1, DH), lambda t, i: (t, 0, 0)),
            pl.BlockSpec((1, 1, DH), lambda t, i: (t, 0, 0)),
        ],
        out_specs=pl.BlockSpec((1, RB, DH), lambda t, i: (osel(t), i, 0)),
        out_shape=jax.ShapeDtypeStruct((2, N, DH), f32),
    )(z, stats, g, be)


def kernel(x_user, x_item, edge_index_ui, edge_index_iu,
           W1r_ui, W1n_ui, b1_ui, W1r_iu, W1n_iu, b1_iu,
           W2r_ui, W2n_ui, b2_ui, W2r_iu, W2n_iu, b2_iu,
           g1_u, be1_u, g1_i, be1_i, g2_u, be2_u, g2_i, be2_i):
    ei_ui = edge_index_ui.astype(jnp.int32)
    ei_iu = edge_index_iu.astype(jnp.int32)

    pad_n = EPS_SUB - E // NS

    def shard(row, fill):
        p = jnp.full((NS, pad_n), fill, jnp.int32)
        return jnp.concatenate([row.reshape(NS, E // NS), p], axis=1)

    src_all = jnp.stack([shard(ei_iu[0], 0), shard(ei_ui[0], 0)])
    src_all = src_all.reshape(NC, NS, NCH, K)
    dst_all = jnp.stack([shard(ei_iu[1], N), shard(ei_ui[1], N)])
    dst_all = dst_all.reshape(NC, NS, NCH, K)

    stk = lambda a, bb: jnp.stack([a, bb])
    row2 = lambda a, bb: jnp.stack([a, bb]).reshape(2, 1, DH)

    NT1 = D // W
    x_flat1 = jnp.concatenate([x_item, x_user], axis=0).reshape(-1, W)
    sc1 = _make_sc_seg(NT1, True, NT1, NT1 * N, 1, 0)(src_all, dst_all,
                                                      x_flat1)

    h1 = _tc_dense(sc1, NT1, sc1, NT1, stk(x_user, x_item), False,
                   stk(W1n_iu, W1n_ui), stk(W1r_iu, W1r_ui),
                   row2(b1_iu, b1_ui), row2(g1_u, g1_i),
                   row2(be1_u, be1_i), True)

    NT2 = DH // W
    sc2 = _make_sc_seg(NT2, False, NT2, NT2 * N, 1, 0)(
        src_all, dst_all, h1.reshape(-1, W))

    h2 = _tc_dense(sc2, NT2, sc1, NT1, h1, True,
                   stk(W2n_iu, W2n_ui), stk(W2r_iu, W2r_ui),
                   row2(b2_iu, b2_ui), row2(g2_u, g2_i),
                   row2(be2_u, be2_i), False)

    return (x_user, x_item, h1[1], h1[0], h2[0], h2[1])


# speedup vs baseline: 1.3703x; 1.0021x over previous
import functools

import jax
import jax.numpy as jnp
from jax import lax
from jax.experimental import pallas as pl
from jax.experimental.pallas import tpu as pltpu
from jax.experimental.pallas import tpu_sc as plsc

N = 10000
D = 128
DH = 256
E = 320000
NC = 2
NS = 16
K = 64
EPS_SUB = 20160
NCH = EPS_SUB // K
NACC = N + 8
FL_W = 640
FL_S = 624
ZR = 16
ZCH = FL_W // ZR
IDXCH = 15
W = 64
NBUF = 5
GLEAD = 3
SLAG = 2
RB = 2000
NG = N // RB


def _fill(ref, rows, cols, value):
    per_row = cols // 16

    def body(i, _):
        r = i // per_row
        c = (i % per_row) * 16
        ref[r, pl.ds(c, 16)] = jnp.full((16,), value, jnp.float32)
        return 0

    lax.fori_loop(0, rows * per_row, body, 0)


def _xform_idx(idxs_v, idxg_v, stride, off):
    per_row = K // 16

    def body(i, _):
        r = i // per_row
        c = (i % per_row) * 16
        idxg_v[r, pl.ds(c, 16)] = idxs_v[r, pl.ds(c, 16)] * stride + off
        return 0

    lax.fori_loop(0, NCH * per_row, body, 0)


def _zero_acc(acc, zb, sid):
    def body(k, _):
        pltpu.sync_copy(zb, acc.at[pl.ds(sid * FL_S + k * ZR, ZR)])
        return 0

    lax.fori_loop(0, ZCH, body, 0)


def _flush_acc(acc, out, cid, t, sid):
    def body(k, _):
        r0 = sid * FL_S + k * ZR
        pltpu.sync_copy(acc.at[pl.ds(r0, ZR)],
                        out.at[cid, t, pl.ds(r0, ZR)])
        return 0

    lax.fori_loop(0, ZCH, body, 0)


def _sc_seg_body(n_tasks, with_counts, stride, core_span, task_span, *refs):
    (src_all, dst_all, table, out, idxs_v, idxd_v, idxg_v, rows) = refs[:8]
    rest = refs[8:]
    if with_counts:
        ones_v, zb, acc = rest[:3]
        rest = rest[3:]
    else:
        zb, acc = rest[:2]
        rest = rest[2:]
        ones_v = None
    sem_g, sem_s = rest[:2]

    cid = lax.axis_index("c")
    sid = lax.axis_index("s")

    _fill(zb, ZR, W, 0.0)
    if with_counts:
        _fill(ones_v, K, W, 1.0)

    def load_idx(k, _):
        r = k * (NCH // IDXCH)
        sl = pl.ds(r, NCH // IDXCH)
        pltpu.sync_copy(src_all.at[cid, sid, sl], idxs_v.at[sl])
        pltpu.sync_copy(dst_all.at[cid, sid, sl], idxd_v.at[sl])
        return 0

    lax.fori_loop(0, IDXCH, load_idx, 0)

    def run_task(t, gather):
        _zero_acc(acc, zb, sid)
        plsc.subcore_barrier()

        if gather:
            _xform_idx(idxs_v, idxg_v, stride,
                       cid * core_span + t * task_span)

            for b in range(GLEAD):
                pltpu.async_copy(table.at[idxg_v.at[b]], rows.at[b],
                                 sem_g.at[b])

            def body(g, _):
                for b in range(NBUF):
                    j = g * NBUF + b
                    pltpu.make_async_copy(table.at[idxg_v.at[j]],
                                          rows.at[b], sem_g.at[b]).wait()
                    pltpu.async_copy(rows.at[b], acc.at[idxd_v.at[j]],
                                     sem_s.at[b], add=True)

                    @pl.when(j >= SLAG)
                    def _(j=j, b=b):
                        b2 = (b - SLAG) % NBUF
                        pltpu.make_async_copy(
                            rows.at[b2], acc.at[idxd_v.at[j - SLAG]],
                            sem_s.at[b2]).wait()

                    @pl.when(j + GLEAD < NCH)
                    def _(j=j, b=b):
                        b3 = (b + GLEAD) % NBUF
                        pltpu.async_copy(table.at[idxg_v.at[j + GLEAD]],
                                         rows.at[b3], sem_g.at[b3])
                return 0

            lax.fori_loop(0, NCH // NBUF, body, 0)
            for jj in range(NCH - SLAG, NCH):
                pltpu.make_async_copy(rows.at[jj % NBUF],
                                      acc.at[idxd_v.at[jj]],
                                      sem_s.at[jj % NBUF]).wait()
        else:
            def body(j, _):
                pltpu.async_copy(ones_v, acc.at[idxd_v.at[j]],
                                 sem_s.at[0], add=True)

                @pl.when(j >= NBUF - 1)
                def _():
                    pltpu.make_async_copy(
                        ones_v, acc.at[idxd_v.at[j]], sem_s.at[0]).wait()
                return 0

            lax.fori_loop(0, NCH, body, 0)

            def drain(j, _):
                pltpu.make_async_copy(ones_v, acc.at[idxd_v.at[j]],
                                      sem_s.at[0]).wait()
                return 0

            lax.fori_loop(0, NBUF - 1, drain, 0)

        plsc.subcore_barrier()
        _flush_acc(acc, out, cid, t, sid)
        plsc.subcore_barrier()

    def task_body(t, _):
        run_task(t, True)
        return 0

    lax.fori_loop(0, n_tasks, task_body, 0)
    if with_counts:
        run_task(n_tasks, False)


def _make_sc_seg(n_tasks, with_counts, stride, core_span, task_span,
                 table_rows):
    f32 = jnp.float32
    n_out = n_tasks + (1 if with_counts else 0)
    scratch = [
        pltpu.VMEM((NCH, K), jnp.int32),
        pltpu.VMEM((NCH, K), jnp.int32),
        pltpu.VMEM((NCH, K), jnp.int32),
    ]
    scratch.append(pltpu.VMEM((NBUF, K, W), f32))
    if with_counts:
        scratch.append(pltpu.VMEM((K, W), f32))
    scratch.append(pltpu.VMEM((ZR, W), f32))
    scratch.append(pltpu.VMEM_SHARED((NACC, W), f32))
    scratch += [pltpu.SemaphoreType.DMA((NBUF,))] * 2
    return pl.kernel(
        functools.partial(_sc_seg_body, n_tasks, with_counts, stride,
                          core_span, task_span),
        out_type=jax.ShapeDtypeStruct((NC, n_out, N, W), f32),
        mesh=plsc.VectorSubcoreMesh(core_axis_name="c", subcore_axis_name="s"),
        compiler_params=pltpu.CompilerParams(use_tc_tiling_on_sc=False),
        scratch_types=scratch,
    )


def _tc_mm_body(n_s, s_ref, cnt_ref, x_ref, wn_ref, wr_ref, b_ref,
                z_ref, stats_ref, acc_ref):
    rcp = 1.0 / jnp.maximum(cnt_ref[0, 0, :, 0:1], 1.0)
    z = jnp.broadcast_to(b_ref[0], (RB, DH))
    for q in range(n_s):
        z = z + jnp.dot(s_ref[0, q] * rcp, wn_ref[0, q * W:(q + 1) * W, :],
                        preferred_element_type=jnp.float32)
    z = z + jnp.dot(x_ref[0], wr_ref[0], preferred_element_type=jnp.float32)
    z_ref[0] = z
    i = pl.program_id(1)

    @pl.when(i == 0)
    def _():
        acc_ref[...] = jnp.zeros_like(acc_ref)

    acc_ref[0:1, :] += jnp.sum(z, axis=0, keepdims=True)
    acc_ref[1:2, :] += jnp.sum(z * z, axis=0, keepdims=True)

    @pl.when(i == NG - 1)
    def _():
        stats_ref[0] = acc_ref[...]


def _tc_bn_body(z_ref, stats_ref, g_ref, be_ref, o_ref):
    m = stats_ref[0, 0:1, :] * (1.0 / N)
    v = stats_ref[0, 1:2, :] * (1.0 / N) - m * m
    z = z_ref[0]
    zn = g_ref[0] * (z - m) / jnp.sqrt(v + 1e-5) + be_ref[0]
    o_ref[0] = jnp.where(zn > 0, zn, jnp.exp(zn) - 1.0)


def _tc_dense(s_all, n_s, cnt_all, cnt_slot, x, x_flip, wn, wr, b, g, be,
              out_flip):
    f32 = jnp.float32
    xw = x.shape[-1]
    xsel = (lambda t: 1 - t) if x_flip else (lambda t: t)
    osel = (lambda t: 1 - t) if out_flip else (lambda t: t)
    z, stats = pl.pallas_call(
        functools.partial(_tc_mm_body, n_s),
        grid=(2, NG),
        in_specs=[
            pl.BlockSpec((1, n_s, RB, W), lambda t, i: (t, 0, i, 0)),
            pl.BlockSpec((1, 1, RB, W), lambda t, i: (t, cnt_slot, i, 0)),
            pl.BlockSpec((1, RB, xw), lambda t, i: (xsel(t), i, 0)),
            pl.BlockSpec((1, n_s * W, DH), lambda t, i: (t, 0, 0)),
            pl.BlockSpec((1, xw, DH), lambda t, i: (t, 0, 0)),
            pl.BlockSpec((1, 1, DH), lambda t, i: (t, 0, 0)),
        ],
        out_specs=(pl.BlockSpec((1, RB, DH), lambda t, i: (t, i, 0)),
                   pl.BlockSpec((1, 2, DH), lambda t, i: (t, 0, 0))),
        out_shape=(jax.ShapeDtypeStruct((2, N, DH), f32),
                   jax.ShapeDtypeStruct((2, 2, DH), f32)),
        scratch_shapes=[pltpu.VMEM((2, DH), f32)],
    )(s_all, cnt_all, x, wn, wr, b)
    return pl.pallas_call(
        _tc_bn_body,
        grid=(2, NG),
        in_specs=[
            pl.BlockSpec((1, RB, DH), lambda t, i: (t, i, 0)),
            pl.BlockSpec((1, 2, DH), lambda t, i: (t, 0, 0)),
            pl.BlockSpec((1, 1, DH), lambda t, i: (t, 0, 0)),
            pl.BlockSpec((1, 1, DH), lambda t, i: (t, 0, 0)),
        ],
        out_specs=pl.BlockSpec((1, RB, DH), lambda t, i: (osel(t), i, 0)),
        out_shape=jax.ShapeDtypeStruct((2, N, DH), f32),
    )(z, stats, g, be)


def kernel(x_user, x_item, edge_index_ui, edge_index_iu,
           W1r_ui, W1n_ui, b1_ui, W1r_iu, W1n_iu, b1_iu,
           W2r_ui, W2n_ui, b2_ui, W2r_iu, W2n_iu, b2_iu,
           g1_u, be1_u, g1_i, be1_i, g2_u, be2_u, g2_i, be2_i):
    ei_ui = edge_index_ui.astype(jnp.int32)
    ei_iu = edge_index_iu.astype(jnp.int32)

    pad_n = EPS_SUB - E // NS

    def shard(row, fill):
        p = jnp.full((NS, pad_n), fill, jnp.int32)
        return jnp.concatenate([row.reshape(NS, E // NS), p], axis=1)

    src_all = jnp.stack([shard(ei_iu[0], 0), shard(ei_ui[0], 0)])
    src_all = src_all.reshape(NC, NS, NCH, K)
    dst_all = jnp.stack([shard(ei_iu[1], N), shard(ei_ui[1], N)])
    dst_all = dst_all.reshape(NC, NS, NCH, K)

    stk = lambda a, bb: jnp.stack([a, bb])
    row2 = lambda a, bb: jnp.stack([a, bb]).reshape(2, 1, DH)

    NT1 = D // W
    x_flat1 = jnp.concatenate([x_item, x_user], axis=0).reshape(-1, W)
    sc1 = _make_sc_seg(NT1, True, NT1, NT1 * N, 1, 0)(src_all, dst_all,
                                                      x_flat1)

    h1 = _tc_dense(sc1, NT1, sc1, NT1, stk(x_user, x_item), False,
                   stk(W1n_iu, W1n_ui), stk(W1r_iu, W1r_ui),
                   row2(b1_iu, b1_ui), row2(g1_u, g1_i),
                   row2(be1_u, be1_i), True)

    NT2 = DH // W
    sc2 = _make_sc_seg(NT2, False, NT2, NT2 * N, 1, 0)(
        src_all, dst_all, h1.reshape(-1, W))

    h2 = _tc_dense(sc2, NT2, sc1, NT1, h1, True,
                   stk(W2n_iu, W2n_ui), stk(W2r_iu, W2r_ui),
                   row2(b2_iu, b2_ui), row2(g2_u, g2_i),
                   row2(be2_u, be2_i), False)

    return (x_user, x_item, h1[1], h1[0], h2[0], h2[1])


# speedup vs baseline: 1.5677x; 1.1441x over previous
import functools

import jax
import jax.numpy as jnp
from jax import lax
from jax.experimental import pallas as pl
from jax.experimental.pallas import tpu as pltpu
from jax.experimental.pallas import tpu_sc as plsc

N = 10000
D = 128
DH = 256
E = 320000
NC = 2
NS = 16
K = 64
EPS_SUB = 20160
NCH = EPS_SUB // K
NACC = N + 8
FL_W = 640
FL_S = 624
ZR = 16
ZCH = FL_W // ZR
IDXCH = 15
W = 64
NBUF = 5
GLEAD = 3
SLAG = 2
RB = 2000
NG = N // RB


def _fill(ref, rows, cols, value):
    per_row = cols // 16

    def body(i, _):
        r = i // per_row
        c = (i % per_row) * 16
        ref[r, pl.ds(c, 16)] = jnp.full((16,), value, jnp.float32)
        return 0

    lax.fori_loop(0, rows * per_row, body, 0)


def _xform_idx(idxs_v, idxg_v, stride, off):
    per_row = K // 16

    def body(i, _):
        r = i // per_row
        c = (i % per_row) * 16
        idxg_v[r, pl.ds(c, 16)] = idxs_v[r, pl.ds(c, 16)] * stride + off
        return 0

    lax.fori_loop(0, NCH * per_row, body, 0)


def _zero_acc(acc, zb, sid, sem):
    def fire(k, _):
        pltpu.async_copy(zb, acc.at[pl.ds(sid * FL_S + k * ZR, ZR)], sem)
        return 0

    def drain(k, _):
        pltpu.make_async_copy(zb, acc.at[pl.ds(sid * FL_S, ZR)],
                              sem).wait()
        return 0

    lax.fori_loop(0, ZCH, fire, 0)
    lax.fori_loop(0, ZCH, drain, 0)


def _flush_acc(acc, out, cid, t, sid, sem):
    def fire(k, _):
        r0 = sid * FL_S + k * ZR
        pltpu.async_copy(acc.at[pl.ds(r0, ZR)],
                         out.at[cid, t, pl.ds(r0, ZR)], sem)
        return 0

    def drain(k, _):
        pltpu.make_async_copy(acc.at[pl.ds(sid * FL_S, ZR)],
                              out.at[cid, t, pl.ds(sid * FL_S, ZR)],
                              sem).wait()
        return 0

    lax.fori_loop(0, ZCH, fire, 0)
    lax.fori_loop(0, ZCH, drain, 0)


def _sc_seg_body(n_tasks, with_counts, stride, core_span, task_span, *refs):
    (src_all, dst_all, table, out, idxs_v, idxd_v, idxg_v, rows) = refs[:8]
    rest = refs[8:]
    if with_counts:
        ones_v, zb, acc = rest[:3]
        rest = rest[3:]
    else:
        zb, acc = rest[:2]
        rest = rest[2:]
        ones_v = None
    sem_g, sem_s = rest[:2]

    cid = lax.axis_index("c")
    sid = lax.axis_index("s")

    _fill(zb, ZR, W, 0.0)
    if with_counts:
        _fill(ones_v, K, W, 1.0)

    def load_idx(k, _):
        r = k * (NCH // IDXCH)
        sl = pl.ds(r, NCH // IDXCH)
        pltpu.sync_copy(src_all.at[cid, sid, sl], idxs_v.at[sl])
        pltpu.sync_copy(dst_all.at[cid, sid, sl], idxd_v.at[sl])
        return 0

    lax.fori_loop(0, IDXCH, load_idx, 0)

    def run_task(t, gather):
        _zero_acc(acc, zb, sid, sem_g.at[0])
        plsc.subcore_barrier()

        if gather:
            _xform_idx(idxs_v, idxg_v, stride,
                       cid * core_span + t * task_span)

            for b in range(GLEAD):
                pltpu.async_copy(table.at[idxg_v.at[b]], rows.at[b],
                                 sem_g.at[b])

            def body(g, _):
                for b in range(NBUF):
                    j = g * NBUF + b
                    pltpu.make_async_copy(table.at[idxg_v.at[j]],
                                          rows.at[b], sem_g.at[b]).wait()
                    pltpu.async_copy(rows.at[b], acc.at[idxd_v.at[j]],
                                     sem_s.at[b], add=True)

                    @pl.when(j >= SLAG)
                    def _(j=j, b=b):
                        b2 = (b - SLAG) % NBUF
                        pltpu.make_async_copy(
                            rows.at[b2], acc.at[idxd_v.at[j - SLAG]],
                            sem_s.at[b2]).wait()

                    @pl.when(j + GLEAD < NCH)
                    def _(j=j, b=b):
                        b3 = (b + GLEAD) % NBUF
                        pltpu.async_copy(table.at[idxg_v.at[j + GLEAD]],
                                         rows.at[b3], sem_g.at[b3])
                return 0

            lax.fori_loop(0, NCH // NBUF, body, 0)
            for jj in range(NCH - SLAG, NCH):
                pltpu.make_async_copy(rows.at[jj % NBUF],
                                      acc.at[idxd_v.at[jj]],
                                      sem_s.at[jj % NBUF]).wait()
        else:
            def body(j, _):
                pltpu.async_copy(ones_v, acc.at[idxd_v.at[j]],
                                 sem_s.at[0], add=True)

                @pl.when(j >= NBUF - 1)
                def _():
                    pltpu.make_async_copy(
                        ones_v, acc.at[idxd_v.at[j]], sem_s.at[0]).wait()
                return 0

            lax.fori_loop(0, NCH, body, 0)

            def drain(j, _):
                pltpu.make_async_copy(ones_v, acc.at[idxd_v.at[j]],
                                      sem_s.at[0]).wait()
                return 0

            lax.fori_loop(0, NBUF - 1, drain, 0)

        plsc.subcore_barrier()
        _flush_acc(acc, out, cid, t, sid, sem_g.at[0])
        plsc.subcore_barrier()

    def task_body(t, _):
        run_task(t, True)
        return 0

    lax.fori_loop(0, n_tasks, task_body, 0)
    if with_counts:
        run_task(n_tasks, False)


def _make_sc_seg(n_tasks, with_counts, stride, core_span, task_span,
                 table_rows):
    f32 = jnp.float32
    n_out = n_tasks + (1 if with_counts else 0)
    scratch = [
        pltpu.VMEM((NCH, K), jnp.int32),
        pltpu.VMEM((NCH, K), jnp.int32),
        pltpu.VMEM((NCH, K), jnp.int32),
    ]
    scratch.append(pltpu.VMEM((NBUF, K, W), f32))
    if with_counts:
        scratch.append(pltpu.VMEM((K, W), f32))
    scratch.append(pltpu.VMEM((ZR, W), f32))
    scratch.append(pltpu.VMEM_SHARED((NACC, W), f32))
    scratch += [pltpu.SemaphoreType.DMA((NBUF,))] * 2
    return pl.kernel(
        functools.partial(_sc_seg_body, n_tasks, with_counts, stride,
                          core_span, task_span),
        out_type=jax.ShapeDtypeStruct((NC, n_out, N, W), f32),
        mesh=plsc.VectorSubcoreMesh(core_axis_name="c", subcore_axis_name="s"),
        compiler_params=pltpu.CompilerParams(use_tc_tiling_on_sc=False),
        scratch_types=scratch,
    )


def _tc_mm_body(n_s, s_ref, cnt_ref, x_ref, wn_ref, wr_ref, b_ref,
                z_ref, stats_ref, acc_ref):
    rcp = 1.0 / jnp.maximum(cnt_ref[0, 0, :, 0:1], 1.0)
    z = jnp.broadcast_to(b_ref[0], (RB, DH))
    for q in range(n_s):
        z = z + jnp.dot(s_ref[0, q] * rcp, wn_ref[0, q * W:(q + 1) * W, :],
                        preferred_element_type=jnp.float32)
    z = z + jnp.dot(x_ref[0], wr_ref[0], preferred_element_type=jnp.float32)
    z_ref[0] = z
    i = pl.program_id(1)

    @pl.when(i == 0)
    def _():
        acc_ref[...] = jnp.zeros_like(acc_ref)

    acc_ref[0:1, :] += jnp.sum(z, axis=0, keepdims=True)
    acc_ref[1:2, :] += jnp.sum(z * z, axis=0, keepdims=True)

    @pl.when(i == NG - 1)
    def _():
        stats_ref[0] = acc_ref[...]


def _tc_bn_body(z_ref, stats_ref, g_ref, be_ref, o_ref):
    m = stats_ref[0, 0:1, :] * (1.0 / N)
    v = stats_ref[0, 1:2, :] * (1.0 / N) - m * m
    z = z_ref[0]
    zn = g_ref[0] * (z - m) / jnp.sqrt(v + 1e-5) + be_ref[0]
    o_ref[0] = jnp.where(zn > 0, zn, jnp.exp(zn) - 1.0)


def _tc_dense(s_all, n_s, cnt_all, cnt_slot, x, x_flip, wn, wr, b, g, be,
              out_flip):
    f32 = jnp.float32
    xw = x.shape[-1]
    xsel = (lambda t: 1 - t) if x_flip else (lambda t: t)
    osel = (lambda t: 1 - t) if out_flip else (lambda t: t)
    z, stats = pl.pallas_call(
        functools.partial(_tc_mm_body, n_s),
        grid=(2, NG),
        in_specs=[
            pl.BlockSpec((1, n_s, RB, W), lambda t, i: (t, 0, i, 0)),
            pl.BlockSpec((1, 1, RB, W), lambda t, i: (t, cnt_slot, i, 0)),
            pl.BlockSpec((1, RB, xw), lambda t, i: (xsel(t), i, 0)),
            pl.BlockSpec((1, n_s * W, DH), lambda t, i: (t, 0, 0)),
            pl.BlockSpec((1, xw, DH), lambda t, i: (t, 0, 0)),
            pl.BlockSpec((1, 1, DH), lambda t, i: (t, 0, 0)),
        ],
        out_specs=(pl.BlockSpec((1, RB, DH), lambda t, i: (t, i, 0)),
                   pl.BlockSpec((1, 2, DH), lambda t, i: (t, 0, 0))),
        out_shape=(jax.ShapeDtypeStruct((2, N, DH), f32),
                   jax.ShapeDtypeStruct((2, 2, DH), f32)),
        scratch_shapes=[pltpu.VMEM((2, DH), f32)],
    )(s_all, cnt_all, x, wn, wr, b)
    return pl.pallas_call(
        _tc_bn_body,
        grid=(2, NG),
        in_specs=[
            pl.BlockSpec((1, RB, DH), lambda t, i: (t, i, 0)),
            pl.BlockSpec((1, 2, DH), lambda t, i: (t, 0, 0)),
            pl.BlockSpec((1, 1, DH), lambda t, i: (t, 0, 0)),
            pl.BlockSpec((1, 1, DH), lambda t, i: (t, 0, 0)),
        ],
        out_specs=pl.BlockSpec((1, RB, DH), lambda t, i: (osel(t), i, 0)),
        out_shape=jax.ShapeDtypeStruct((2, N, DH), f32),
    )(z, stats, g, be)


def kernel(x_user, x_item, edge_index_ui, edge_index_iu,
           W1r_ui, W1n_ui, b1_ui, W1r_iu, W1n_iu, b1_iu,
           W2r_ui, W2n_ui, b2_ui, W2r_iu, W2n_iu, b2_iu,
           g1_u, be1_u, g1_i, be1_i, g2_u, be2_u, g2_i, be2_i):
    ei_ui = edge_index_ui.astype(jnp.int32)
    ei_iu = edge_index_iu.astype(jnp.int32)

    pad_n = EPS_SUB - E // NS

    def shard(row, fill):
        p = jnp.full((NS, pad_n), fill, jnp.int32)
        return jnp.concatenate([row.reshape(NS, E // NS), p], axis=1)

    src_all = jnp.stack([shard(ei_iu[0], 0), shard(ei_ui[0], 0)])
    src_all = src_all.reshape(NC, NS, NCH, K)
    dst_all = jnp.stack([shard(ei_iu[1], N), shard(ei_ui[1], N)])
    dst_all = dst_all.reshape(NC, NS, NCH, K)

    stk = lambda a, bb: jnp.stack([a, bb])
    row2 = lambda a, bb: jnp.stack([a, bb]).reshape(2, 1, DH)

    NT1 = D // W
    x_flat1 = jnp.concatenate([x_item, x_user], axis=0).reshape(-1, W)
    sc1 = _make_sc_seg(NT1, True, NT1, NT1 * N, 1, 0)(src_all, dst_all,
                                                      x_flat1)

    h1 = _tc_dense(sc1, NT1, sc1, NT1, stk(x_user, x_item), False,
                   stk(W1n_iu, W1n_ui), stk(W1r_iu, W1r_ui),
                   row2(b1_iu, b1_ui), row2(g1_u, g1_i),
                   row2(be1_u, be1_i), True)

    NT2 = DH // W
    sc2 = _make_sc_seg(NT2, False, NT2, NT2 * N, 1, 0)(
        src_all, dst_all, h1.reshape(-1, W))

    h2 = _tc_dense(sc2, NT2, sc1, NT1, h1, True,
                   stk(W2n_iu, W2n_ui), stk(W2r_iu, W2r_ui),
                   row2(b2_iu, b2_ui), row2(g2_u, g2_i),
                   row2(be2_u, be2_i), False)

    return (x_user, x_item, h1[1], h1[0], h2[0], h2[1])
